# Initial kernel scaffold; baseline (speedup 1.0000x reference)
import jax
import jax.numpy as jnp
from jax.experimental import pallas as pl


def kernel(bin_index, table):
    raise NotImplementedError("write your pallas kernel here")


# speedup vs baseline: 1.2714x; 1.2714x over previous
import functools

import jax
import jax.numpy as jnp
from jax import lax
from jax.experimental import pallas as pl
from jax.experimental.pallas import tpu as pltpu
from jax.experimental.pallas import tpu_sc as plsc

NUM_BINS = 64
EMBED_DIM = 64

_NC = 2
_NS = 16
_NW = _NC * _NS

_B = 8 * 100000
_PER_W = _B // _NW
_CHUNK = 1000
_NCHUNK = _PER_W // _CHUNK


def _sc_embed(idx_hbm, table_hbm, out_hbm,
              idx0, idx1, rows0, rows1,
              sem_i0, sem_i1, sem_g0, sem_g1, sem_o0, sem_o1):
    c = lax.axis_index("c")
    s = lax.axis_index("s")
    wid = s * _NC + c
    base = wid * _PER_W

    idx = (idx0, idx1)
    rows = (rows0, rows1)
    sem_i = (sem_i0, sem_i1)
    sem_g = (sem_g0, sem_g1)
    sem_o = (sem_o0, sem_o1)

    loads = [None, None]
    stores = [None, None]

    for i in range(min(2, _NCHUNK)):
        loads[i] = pltpu.async_copy(
            idx_hbm.at[pl.ds(base + i * _CHUNK, _CHUNK)], idx[i], sem_i[i])

    for i in range(_NCHUNK):
        sl = i % 2
        loads[sl].wait()
        if stores[sl] is not None:
            stores[sl].wait()
        g = pltpu.async_copy(table_hbm.at[idx[sl]], rows[sl], sem_g[sl])
        g.wait()
        stores[sl] = pltpu.async_copy(
            rows[sl], out_hbm.at[pl.ds(base + i * _CHUNK, _CHUNK)], sem_o[sl])
        if i + 2 < _NCHUNK:
            loads[sl] = pltpu.async_copy(
                idx_hbm.at[pl.ds(base + (i + 2) * _CHUNK, _CHUNK)],
                idx[sl], sem_i[sl])

    for sl in range(2):
        if stores[sl] is not None:
            stores[sl].wait()


@jax.jit
def _run(idx_flat, table):
    mesh = plsc.VectorSubcoreMesh(core_axis_name="c", subcore_axis_name="s")
    k = functools.partial(
        pl.kernel,
        out_type=jax.ShapeDtypeStruct((_B, EMBED_DIM), jnp.float32),
        mesh=mesh,
        compiler_params=pltpu.CompilerParams(use_tc_tiling_on_sc=False),
        scratch_types=[
            pltpu.VMEM((_CHUNK,), jnp.int32),
            pltpu.VMEM((_CHUNK,), jnp.int32),
            pltpu.VMEM((_CHUNK, EMBED_DIM), jnp.float32),
            pltpu.VMEM((_CHUNK, EMBED_DIM), jnp.float32),
            pltpu.SemaphoreType.DMA,
            pltpu.SemaphoreType.DMA,
            pltpu.SemaphoreType.DMA,
            pltpu.SemaphoreType.DMA,
            pltpu.SemaphoreType.DMA,
            pltpu.SemaphoreType.DMA,
        ],
    )(_sc_embed)
    return k(idx_flat, table)


def kernel(bin_index, table):
    idx_flat = bin_index.reshape(-1).astype(jnp.int32)
    out = _run(idx_flat, table)
    return out.reshape(8, 100000, EMBED_DIM)


# speedup vs baseline: 2.6749x; 2.1039x over previous
import functools

import jax
import jax.numpy as jnp
from jax import lax
from jax.experimental import pallas as pl
from jax.experimental.pallas import tpu as pltpu
from jax.experimental.pallas import tpu_sc as plsc

NUM_BINS = 64
EMBED_DIM = 64

_NC = 2
_NS = 16
_NW = _NC * _NS

_B = 8 * 100000
_PER_W = _B // _NW
_CHUNK = 1000
_NCHUNK = _PER_W // _CHUNK


def _sc_embed(idx_hbm, table_hbm, out_hbm,
              idx0, idx1, rows0, rows1,
              sem_i0, sem_i1, sem_g0, sem_g1, sem_o0, sem_o1):
    c = lax.axis_index("c")
    s = lax.axis_index("s")
    wid = s * _NC + c
    base = wid * _PER_W

    idx = (idx0, idx1)
    rows = (rows0, rows1)
    sem_i = (sem_i0, sem_i1)
    sem_g = (sem_g0, sem_g1)
    sem_o = (sem_o0, sem_o1)

    loads = [None, None]
    stores = [None, None]

    for i in range(min(2, _NCHUNK)):
        loads[i] = pltpu.async_copy(
            idx_hbm.at[pl.ds(base + i * _CHUNK, _CHUNK)], idx[i], sem_i[i])

    for i in range(_NCHUNK):
        sl = i % 2
        loads[sl].wait()
        if stores[sl] is not None:
            stores[sl].wait()
        g = pltpu.async_copy(table_hbm.at[idx[sl]], rows[sl], sem_g[sl])
        g.wait()
        stores[sl] = pltpu.async_copy(
            rows[sl], out_hbm.at[pl.ds(base + i * _CHUNK, _CHUNK)], sem_o[sl])
        if i + 2 < _NCHUNK:
            loads[sl] = pltpu.async_copy(
                idx_hbm.at[pl.ds(base + (i + 2) * _CHUNK, _CHUNK)],
                idx[sl], sem_i[sl])

    for sl in range(2):
        if stores[sl] is not None:
            stores[sl].wait()


@jax.jit
def _run(idx_flat, table):
    mesh = plsc.VectorSubcoreMesh(core_axis_name="c", subcore_axis_name="s")
    k = functools.partial(
        pl.kernel,
        out_type=jax.ShapeDtypeStruct((_B, EMBED_DIM), jnp.float32),
        mesh=mesh,
        compiler_params=pltpu.CompilerParams(use_tc_tiling_on_sc=False),
        scratch_types=[
            pltpu.VMEM((_CHUNK,), jnp.int32),
            pltpu.VMEM((_CHUNK,), jnp.int32),
            pltpu.VMEM((_CHUNK, EMBED_DIM), jnp.float32),
            pltpu.VMEM((_CHUNK, EMBED_DIM), jnp.float32),
            pltpu.SemaphoreType.DMA,
            pltpu.SemaphoreType.DMA,
            pltpu.SemaphoreType.DMA,
            pltpu.SemaphoreType.DMA,
            pltpu.SemaphoreType.DMA,
            pltpu.SemaphoreType.DMA,
        ],
    )(_sc_embed)
    return k(idx_flat, table)


def kernel(bin_index, table):
    idx_flat = bin_index.reshape(-1).astype(jnp.int32)
    table_rep = jnp.broadcast_to(table[None], (_NW, NUM_BINS, EMBED_DIM))
    table_rep = table_rep.reshape(_NW * NUM_BINS, EMBED_DIM)
    offs = jnp.repeat(jnp.arange(_NW, dtype=jnp.int32) * NUM_BINS, _PER_W)
    out = _run(idx_flat + offs, table_rep)
    return out.reshape(8, 100000, EMBED_DIM)


# speedup vs baseline: 2.6785x; 1.0013x over previous
import functools

import jax
import jax.numpy as jnp
from jax import lax
from jax.experimental import pallas as pl
from jax.experimental.pallas import tpu as pltpu
from jax.experimental.pallas import tpu_sc as plsc

NUM_BINS = 64
EMBED_DIM = 64

_NC = 2
_NS = 16
_NW = _NC * _NS

_B = 8 * 100000
_PER_W = _B // _NW
_CHUNK = 1000
_NCHUNK = _PER_W // _CHUNK


def _sc_embed(idx_hbm, table_hbm, out_hbm,
              idx0, idx1, rows0, rows1,
              sem_i0, sem_i1, sem_g0, sem_g1, sem_o0, sem_o1):
    c = lax.axis_index("c")
    s = lax.axis_index("s")
    wid = s * _NC + c
    base = wid * _PER_W
    out_b = wid // (100000 // _PER_W)
    out_r = (wid % (100000 // _PER_W)) * _PER_W

    idx = (idx0, idx1)
    rows = (rows0, rows1)
    sem_i = (sem_i0, sem_i1)
    sem_g = (sem_g0, sem_g1)
    sem_o = (sem_o0, sem_o1)

    loads = [None, None]
    stores = [None, None]

    for i in range(min(2, _NCHUNK)):
        loads[i] = pltpu.async_copy(
            idx_hbm.at[pl.ds(base + i * _CHUNK, _CHUNK)], idx[i], sem_i[i])

    for i in range(_NCHUNK):
        sl = i % 2
        loads[sl].wait()
        if stores[sl] is not None:
            stores[sl].wait()
        g = pltpu.async_copy(table_hbm.at[idx[sl]], rows[sl], sem_g[sl])
        g.wait()
        stores[sl] = pltpu.async_copy(
            rows[sl],
            out_hbm.at[out_b, pl.ds(out_r + i * _CHUNK, _CHUNK)],
            sem_o[sl])
        if i + 2 < _NCHUNK:
            loads[sl] = pltpu.async_copy(
                idx_hbm.at[pl.ds(base + (i + 2) * _CHUNK, _CHUNK)],
                idx[sl], sem_i[sl])

    for sl in range(2):
        if stores[sl] is not None:
            stores[sl].wait()


@jax.jit
def _run(idx_flat, table):
    mesh = plsc.VectorSubcoreMesh(core_axis_name="c", subcore_axis_name="s")
    k = functools.partial(
        pl.kernel,
        out_type=jax.ShapeDtypeStruct((8, 100000, EMBED_DIM), jnp.float32),
        mesh=mesh,
        compiler_params=pltpu.CompilerParams(use_tc_tiling_on_sc=False),
        scratch_types=[
            pltpu.VMEM((_CHUNK,), jnp.int32),
            pltpu.VMEM((_CHUNK,), jnp.int32),
            pltpu.VMEM((_CHUNK, EMBED_DIM), jnp.float32),
            pltpu.VMEM((_CHUNK, EMBED_DIM), jnp.float32),
            pltpu.SemaphoreType.DMA,
            pltpu.SemaphoreType.DMA,
            pltpu.SemaphoreType.DMA,
            pltpu.SemaphoreType.DMA,
            pltpu.SemaphoreType.DMA,
            pltpu.SemaphoreType.DMA,
        ],
    )(_sc_embed)
    return k(idx_flat, table)


def kernel(bin_index, table):
    idx_flat = bin_index.reshape(-1).astype(jnp.int32)
    table_rep = jnp.broadcast_to(table[None], (_NW, NUM_BINS, EMBED_DIM))
    table_rep = table_rep.reshape(_NW * NUM_BINS, EMBED_DIM)
    offs = jnp.repeat(jnp.arange(_NW, dtype=jnp.int32) * NUM_BINS, _PER_W)
    return _run(idx_flat + offs, table_rep)


# speedup vs baseline: 3.3017x; 1.2327x over previous
import functools

import jax
import jax.numpy as jnp
from jax import lax
from jax.experimental import pallas as pl
from jax.experimental.pallas import tpu as pltpu
from jax.experimental.pallas import tpu_sc as plsc

NUM_BINS = 64
EMBED_DIM = 64
_TAB_W = 128

_NC = 2
_NS = 16
_NW = _NC * _NS

_N = 100000
_NPAD = 100096
_W_PER_ROW = 4
_STRIPE = 25088
_CHUNK = 896
_NFULL = 27
_LAST = _STRIPE - _NFULL * _CHUNK
_TAIL = _NPAD - 3 * _STRIPE - _NFULL * _CHUNK
_TAIL_REAL = _N - 3 * _STRIPE - _NFULL * _CHUNK


def _gather_group(tab, idx_ref, buf, off):
    ivec = idx_ref[pl.ds(off, 16)]

    def dstep(d, addr):
        buf[d, pl.ds(off, 16)] = plsc.load_gather(tab, [addr])
        return addr + _TAB_W

    lax.fori_loop(0, EMBED_DIM, dstep, ivec, unroll=8)


def _compute_chunk(tab, idx_ref, buf, n):
    def ngroup(g, _):
        _gather_group(tab, idx_ref, buf, g * 16)
        return 0

    lax.fori_loop(0, n // 16, ngroup, 0)


def _sc_embed(idx_hbm, tab_hbm, out_hbm,
              tab_v, idx0, idx1, buf0, buf1,
              sem_t, sem_i0, sem_i1, sem_o0, sem_o1):
    c = lax.axis_index("c")
    s = lax.axis_index("s")
    wid = s * _NC + c
    out_b = wid // _W_PER_ROW
    t = wid % _W_PER_ROW
    out_n0 = t * _STRIPE
    base = out_b * _N + out_n0

    idx = (idx0, idx1)
    buf = (buf0, buf1)
    sem_i = (sem_i0, sem_i1)
    sem_o = (sem_o0, sem_o1)

    tload = pltpu.async_copy(tab_hbm, tab_v, sem_t)
    loads = [None, None]
    stores = [None, None]
    for i in range(2):
        loads[i] = pltpu.async_copy(
            idx_hbm.at[pl.ds(base + i * _CHUNK, _CHUNK)], idx[i], sem_i[i])
    tload.wait()

    for i in range(_NFULL):
        sl = i % 2
        loads[sl].wait()
        if stores[sl] is not None:
            stores[sl].wait()
        _compute_chunk(tab_v, idx[sl], buf[sl], _CHUNK)
        stores[sl] = pltpu.async_copy(
            buf[sl],
            out_hbm.at[out_b, :, pl.ds(out_n0 + i * _CHUNK, _CHUNK)],
            sem_o[sl])
        if i + 2 < _NFULL:
            loads[sl] = pltpu.async_copy(
                idx_hbm.at[pl.ds(base + (i + 2) * _CHUNK, _CHUNK)],
                idx[sl], sem_i[sl])

    for sl in range(2):
        stores[sl].wait()

    tail_off = _NFULL * _CHUNK

    @pl.when(t < _W_PER_ROW - 1)
    def _():
        pltpu.async_copy(
            idx_hbm.at[pl.ds(base + tail_off, _LAST)], idx0, sem_i0).wait()
        _compute_chunk(tab_v, idx0, buf0, _LAST)
        pltpu.async_copy(
            buf0,
            out_hbm.at[out_b, :, pl.ds(out_n0 + tail_off, _LAST)],
            sem_o0).wait()

    @pl.when(t == _W_PER_ROW - 1)
    def _():
        pltpu.async_copy(
            idx_hbm.at[pl.ds(base + tail_off, _TAIL_REAL)],
            idx1.at[pl.ds(0, _TAIL_REAL)], sem_i1).wait()
        zeros = jnp.zeros((16,), jnp.int32)
        for z in range(_TAIL_REAL, _TAIL, 16):
            idx1[pl.ds(z, 16)] = zeros
        _compute_chunk(tab_v, idx1, buf1, _TAIL)
        pltpu.async_copy(
            buf1.at[:, pl.ds(0, _TAIL)],
            out_hbm.at[out_b, :, pl.ds(out_n0 + tail_off, _TAIL)],
            sem_o1).wait()


@jax.jit
def _run(idx_flat, tab_t):
    mesh = plsc.VectorSubcoreMesh(core_axis_name="c", subcore_axis_name="s")
    k = functools.partial(
        pl.kernel,
        out_type=jax.ShapeDtypeStruct((8, EMBED_DIM, _NPAD), jnp.float32),
        mesh=mesh,
        compiler_params=pltpu.CompilerParams(needs_layout_passes=False),
        scratch_types=[
            pltpu.VMEM((EMBED_DIM * _TAB_W,), jnp.float32),
            pltpu.VMEM((_CHUNK,), jnp.int32),
            pltpu.VMEM((_CHUNK,), jnp.int32),
            pltpu.VMEM((EMBED_DIM, _CHUNK), jnp.float32),
            pltpu.VMEM((EMBED_DIM, _CHUNK), jnp.float32),
            pltpu.SemaphoreType.DMA,
            pltpu.SemaphoreType.DMA,
            pltpu.SemaphoreType.DMA,
            pltpu.SemaphoreType.DMA,
            pltpu.SemaphoreType.DMA,
        ],
    )(_sc_embed)
    return k(idx_flat, tab_t)


def kernel(bin_index, table):
    idx_flat = bin_index.reshape(-1).astype(jnp.int32)
    tab_t = jnp.pad(table.T, ((0, 0), (0, _TAB_W - NUM_BINS))).reshape(-1)
    out_t = _run(idx_flat, tab_t)
    return jnp.transpose(out_t[:, :, :_N], (0, 2, 1))


# speedup vs baseline: 3.4153x; 1.0344x over previous
import functools

import jax
import jax.numpy as jnp
from jax import lax
from jax.experimental import pallas as pl
from jax.experimental.pallas import tpu as pltpu
from jax.experimental.pallas import tpu_sc as plsc

NUM_BINS = 64
EMBED_DIM = 64
_TAB_W = 128

_NC = 2
_NS = 16
_NW = _NC * _NS

_N = 100000
_NPAD = 100096
_W_PER_ROW = 4
_STRIPE = 25088
_CHUNK = 896
_NFULL = 27
_LAST = _STRIPE - _NFULL * _CHUNK
_TAIL = _NPAD - 3 * _STRIPE - _NFULL * _CHUNK
_TAIL_REAL = _N - 3 * _STRIPE - _NFULL * _CHUNK


def _gather_group(tab, idx_ref, buf, off):
    addr = idx_ref[pl.ds(off, 16)]
    for d in range(EMBED_DIM):
        buf[d, pl.ds(off, 16)] = plsc.load_gather(tab, [addr])
        if d + 1 < EMBED_DIM:
            addr = addr + _TAB_W


def _compute_chunk(tab, idx_ref, buf, n):
    def ngroup(g, _):
        _gather_group(tab, idx_ref, buf, g * 16)
        return 0

    lax.fori_loop(0, n // 16, ngroup, 0)


def _sc_embed(idx_hbm, tab_hbm, out_hbm,
              tab_v, idx0, idx1, buf0, buf1,
              sem_t, sem_i0, sem_i1, sem_o0, sem_o1):
    c = lax.axis_index("c")
    s = lax.axis_index("s")
    wid = s * _NC + c
    out_b = wid // _W_PER_ROW
    t = wid % _W_PER_ROW
    out_n0 = t * _STRIPE
    base = out_b * _N + out_n0

    idx = (idx0, idx1)
    buf = (buf0, buf1)
    sem_i = (sem_i0, sem_i1)
    sem_o = (sem_o0, sem_o1)

    tload = pltpu.async_copy(tab_hbm, tab_v, sem_t)
    loads = [None, None]
    stores = [None, None]
    for i in range(2):
        loads[i] = pltpu.async_copy(
            idx_hbm.at[pl.ds(base + i * _CHUNK, _CHUNK)], idx[i], sem_i[i])
    tload.wait()

    for i in range(_NFULL):
        sl = i % 2
        loads[sl].wait()
        if stores[sl] is not None:
            stores[sl].wait()
        _compute_chunk(tab_v, idx[sl], buf[sl], _CHUNK)
        stores[sl] = pltpu.async_copy(
            buf[sl],
            out_hbm.at[out_b, :, pl.ds(out_n0 + i * _CHUNK, _CHUNK)],
            sem_o[sl])
        if i + 2 < _NFULL:
            loads[sl] = pltpu.async_copy(
                idx_hbm.at[pl.ds(base + (i + 2) * _CHUNK, _CHUNK)],
                idx[sl], sem_i[sl])

    for sl in range(2):
        stores[sl].wait()

    tail_off = _NFULL * _CHUNK

    @pl.when(t < _W_PER_ROW - 1)
    def _():
        pltpu.async_copy(
            idx_hbm.at[pl.ds(base + tail_off, _LAST)], idx0, sem_i0).wait()
        _compute_chunk(tab_v, idx0, buf0, _LAST)
        pltpu.async_copy(
            buf0,
            out_hbm.at[out_b, :, pl.ds(out_n0 + tail_off, _LAST)],
            sem_o0).wait()

    @pl.when(t == _W_PER_ROW - 1)
    def _():
        pltpu.async_copy(
            idx_hbm.at[pl.ds(base + tail_off, _TAIL_REAL)],
            idx1.at[pl.ds(0, _TAIL_REAL)], sem_i1).wait()
        zeros = jnp.zeros((16,), jnp.int32)
        for z in range(_TAIL_REAL, _TAIL, 16):
            idx1[pl.ds(z, 16)] = zeros
        _compute_chunk(tab_v, idx1, buf1, _TAIL)
        pltpu.async_copy(
            buf1.at[:, pl.ds(0, _TAIL)],
            out_hbm.at[out_b, :, pl.ds(out_n0 + tail_off, _TAIL)],
            sem_o1).wait()


@jax.jit
def _run(idx_flat, tab_t):
    mesh = plsc.VectorSubcoreMesh(core_axis_name="c", subcore_axis_name="s")
    k = functools.partial(
        pl.kernel,
        out_type=jax.ShapeDtypeStruct((8, EMBED_DIM, _NPAD), jnp.float32),
        mesh=mesh,
        compiler_params=pltpu.CompilerParams(needs_layout_passes=False),
        scratch_types=[
            pltpu.VMEM((EMBED_DIM * _TAB_W,), jnp.float32),
            pltpu.VMEM((_CHUNK,), jnp.int32),
            pltpu.VMEM((_CHUNK,), jnp.int32),
            pltpu.VMEM((EMBED_DIM, _CHUNK), jnp.float32),
            pltpu.VMEM((EMBED_DIM, _CHUNK), jnp.float32),
            pltpu.SemaphoreType.DMA,
            pltpu.SemaphoreType.DMA,
            pltpu.SemaphoreType.DMA,
            pltpu.SemaphoreType.DMA,
            pltpu.SemaphoreType.DMA,
        ],
    )(_sc_embed)
    return k(idx_flat, tab_t)


def kernel(bin_index, table):
    idx_flat = bin_index.reshape(-1).astype(jnp.int32)
    tab_t = jnp.pad(table.T, ((0, 0), (0, _TAB_W - NUM_BINS))).reshape(-1)
    out_t = _run(idx_flat, tab_t)
    return jnp.transpose(out_t[:, :, :_N], (0, 2, 1))
